# baseline (speedup 1.0000x reference)
import functools

import jax
import jax.numpy as jnp
from jax import lax
from jax.experimental import pallas as pl
from jax.experimental.pallas import tpu as pltpu
from jax.experimental.pallas import tpu_sc as plsc

VOCAB = 1000000
DIM = 64
NUM_CLASSES = 2
B = 4096
L = 200

NC = 2
NS = 16
NW = NC * NS
BPW = B // NW
LANES = 16
NV = DIM // LANES

CHUNKS = ((0, 128), (128, 72))

_mesh = plsc.VectorSubcoreMesh(core_axis_name="c", subcore_axis_name="s")


@functools.partial(
    pl.kernel,
    mesh=_mesh,
    compiler_params=pltpu.CompilerParams(use_tc_tiling_on_sc=False),
    out_type=jax.ShapeDtypeStruct((B, DIM), jnp.float32),
    scratch_types=[
        pltpu.VMEM((BPW, L), jnp.int32),
        pltpu.VMEM((L, DIM), jnp.float32),
        pltpu.VMEM((BPW, DIM), jnp.float32),
        pltpu.SemaphoreType.DMA,
    ],
)
def _pool_sums(x_hbm, table_hbm, out_hbm, idx_v, rows_v, acc_v, sem):
    wid = lax.axis_index("s") * NC + lax.axis_index("c")
    base = wid * BPW
    pltpu.sync_copy(x_hbm.at[pl.ds(base, BPW)], idx_v)

    def row_body(r, _):
        cps = []
        for off, n in CHUNKS:
            cps.append(
                pltpu.async_copy(
                    table_hbm.at[idx_v.at[r, pl.ds(off, n)]],
                    rows_v.at[pl.ds(off, n)],
                    sem,
                )
            )
        for cp in cps:
            cp.wait()

        def accum(j, accs):
            return tuple(
                accs[k] + rows_v[j, pl.ds(k * LANES, LANES)] for k in range(NV)
            )

        zero = jnp.zeros((LANES,), jnp.float32)
        accs = lax.fori_loop(0, L, accum, (zero,) * NV, unroll=4)
        for k in range(NV):
            acc_v[r, pl.ds(k * LANES, LANES)] = accs[k]
        return 0

    lax.fori_loop(0, BPW, row_body, 0)
    pltpu.sync_copy(acc_v, out_hbm.at[pl.ds(base, BPW)])


def _classifier_body(s_ref, w_ref, b_ref, o_ref):
    p = s_ref[...] * (1.0 / L)
    logits = lax.dot_general(
        p, w_ref[...], (((1,), (1,)), ((), ())),
        preferred_element_type=jnp.float32,
    ) + b_ref[...]
    m = jnp.max(logits, axis=-1, keepdims=True)
    e = jnp.exp(logits - m)
    o_ref[...] = e / jnp.sum(e, axis=-1, keepdims=True)


def kernel(x, table, W_w, W_b):
    sums = _pool_sums(x, table)
    return pl.pallas_call(
        _classifier_body,
        out_shape=jax.ShapeDtypeStruct((B, NUM_CLASSES), jnp.float32),
    )(sums, W_w, W_b.reshape(1, NUM_CLASSES))
